# Initial kernel scaffold; baseline (speedup 1.0000x reference)
import functools
import math

import jax
import jax.numpy as jnp
from jax import lax
from jax.experimental import pallas as pl
from jax.experimental.pallas import tpu as pltpu
import jax.experimental.pallas.tpu_sc as plsc

_N = 100000
_G = 1024
_T_MAX = 1000
_D = 128
_HALF = _D // 2

_NC = 2
_NS = 16
_NW = _NC * _NS
_BLK = 128
_NBLK = _N // _BLK
_TAIL = _N - _NBLK * _BLK
_EXTRA = _NBLK - 24 * _NW
_TAIL_OFF = _NBLK * _BLK


def _tables_body(t_ref, alpha_ref, sigma_ref, emb_ref):
    tf = t_ref[...].astype(jnp.float32)
    ang = tf * (0.5 * math.pi / _T_MAX)
    alpha_ref[...] = jnp.cos(ang)
    sigma_ref[...] = jnp.sin(ang)
    j = lax.broadcasted_iota(jnp.float32, (_G, _HALF), 1)
    freqs = jnp.exp(j * (-math.log(10000.0) / _HALF))
    args = tf * freqs
    emb_ref[:, :_HALF] = jnp.sin(args)
    emb_ref[:, _HALF:] = jnp.cos(args)


def _tables(t2):
    return pl.pallas_call(
        _tables_body,
        out_shape=(
            jax.ShapeDtypeStruct((_G, 1), jnp.float32),
            jax.ShapeDtypeStruct((_G, 1), jnp.float32),
            jax.ShapeDtypeStruct((_G, _D), jnp.float32),
        ),
    )(t2)


def _combine_body(p_ref, means_ref):
    p = p_ref[0] + p_ref[1]
    cnt = jnp.maximum(p[:, 6:7], 1.0)
    means_ref[...] = p / cnt


def _combine(partials):
    return pl.pallas_call(
        _combine_body,
        out_shape=jax.ShapeDtypeStruct((_G, 8), jnp.float32),
    )(partials)


def _worker_id():
    return lax.axis_index("s") * _NC + lax.axis_index("c")


def _chunk_of(wid):
    nb = 24 + jnp.where(wid < _EXTRA, 1, 0)
    base_blk = 24 * wid + jnp.minimum(wid, _EXTRA)
    return nb, base_blk * _BLK


def _iota16():
    return lax.iota(jnp.int32, 16)


def _fill_vals(n, idx_v, pbuf, ebuf, vals):
    iota = _iota16()
    ones = jnp.ones((16,), jnp.float32)
    for k in range(n // 16):
        rv = iota + k * 16
        rv3 = rv * 3
        for base, buf in ((0, pbuf), (3, ebuf)):
            for c in range(3):
                v = plsc.load_gather(buf, [rv3 + c])
                plsc.store_scatter(vals, [rv, jnp.full((16,), base + c, jnp.int32)], v)
        plsc.store_scatter(vals, [rv, jnp.full((16,), 6, jnp.int32)], ones)


def _zero_col7(n, vals):
    iota = _iota16()
    zeros = jnp.zeros((16,), jnp.float32)
    for k in range(n // 16):
        rv = iota + k * 16
        plsc.store_scatter(vals, [rv, jnp.full((16,), 7, jnp.int32)], zeros)


def _sums_body(posf, epsf, batch, out, acc, idx_v, pbuf, ebuf, vals,
               idx_t, pbuf_t, ebuf_t, vals_t, zbuf, obuf):
    c = lax.axis_index("c")
    s = lax.axis_index("s")
    wid = _worker_id()

    iota = _iota16()
    zeros = jnp.zeros((16,), jnp.float32)
    for i in range(32):
        rv = i * 2 + iota // 8
        cv = iota % 8
        plsc.store_scatter(zbuf, [rv, cv], zeros)
    pltpu.sync_copy(zbuf, acc.at[pl.ds(s * 64, 64), :])
    plsc.subcore_barrier()

    _zero_col7(_BLK, vals)
    nb, rb = _chunk_of(wid)

    def blk(j, carry):
        r0 = rb + j * _BLK
        pltpu.sync_copy(batch.at[pl.ds(r0, _BLK)], idx_v)
        pltpu.sync_copy(posf.at[pl.ds(r0 * 3, _BLK * 3)], pbuf)
        pltpu.sync_copy(epsf.at[pl.ds(r0 * 3, _BLK * 3)], ebuf)
        _fill_vals(_BLK, idx_v, pbuf, ebuf, vals)
        pltpu.sync_copy(vals, acc.at[idx_v], add=True)
        return carry

    lax.fori_loop(0, nb, blk, 0)

    @pl.when(wid == _NW - 1)
    def _tail():
        r0 = _TAIL_OFF
        pltpu.sync_copy(batch.at[pl.ds(r0, _TAIL)], idx_t)
        pltpu.sync_copy(posf.at[pl.ds(r0 * 3, _TAIL * 3)], pbuf_t)
        pltpu.sync_copy(epsf.at[pl.ds(r0 * 3, _TAIL * 3)], ebuf_t)
        _zero_col7(_TAIL, vals_t)
        _fill_vals(_TAIL, idx_t, pbuf_t, ebuf_t, vals_t)
        pltpu.sync_copy(vals_t, acc.at[idx_t], add=True)

    plsc.subcore_barrier()
    pltpu.sync_copy(acc.at[pl.ds(s * 64, 64), :], obuf)
    pltpu.sync_copy(obuf, out.at[c, pl.ds(s * 64, 64), :])


def _sc_sums(posf, epsf, batch):
    mesh = plsc.VectorSubcoreMesh(core_axis_name="c", subcore_axis_name="s")
    f = pl.kernel(
        _sums_body,
        out_type=jax.ShapeDtypeStruct((_NC, _G, 8), jnp.float32),
        mesh=mesh,
        scratch_types=[
            pltpu.VMEM_SHARED((_G, 8), jnp.float32),
            pltpu.VMEM((_BLK,), jnp.int32),
            pltpu.VMEM((_BLK * 3,), jnp.float32),
            pltpu.VMEM((_BLK * 3,), jnp.float32),
            pltpu.VMEM((_BLK, 8), jnp.float32),
            pltpu.VMEM((_TAIL,), jnp.int32),
            pltpu.VMEM((_TAIL * 3,), jnp.float32),
            pltpu.VMEM((_TAIL * 3,), jnp.float32),
            pltpu.VMEM((_TAIL, 8), jnp.float32),
            pltpu.VMEM((64, 8), jnp.float32),
            pltpu.VMEM((64, 8), jnp.float32),
        ],
    )
    return f(posf, epsf, batch)


def _center_block(n, idx_v, pbuf, ebuf, alpha_v, sigma_v, mean_v, pobuf, eobuf):
    iota = _iota16()
    for k in range(n // 16):
        rv = iota + k * 16
        rv3 = rv * 3
        g = idx_v[pl.ds(k * 16, 16)]
        a = plsc.load_gather(alpha_v, [g])
        sg = plsc.load_gather(sigma_v, [g])
        for c in range(3):
            px = plsc.load_gather(pbuf, [rv3 + c])
            ex = plsc.load_gather(ebuf, [rv3 + c])
            mp = plsc.load_gather(mean_v, [g, jnp.full((16,), c, jnp.int32)])
            me = plsc.load_gather(mean_v, [g, jnp.full((16,), 3 + c, jnp.int32)])
            x = px - mp
            e = ex - me
            plsc.store_scatter(eobuf, [rv3 + c], e)
            plsc.store_scatter(pobuf, [rv3 + c], a * x + sg * e)


def _main_body(posf, epsf, batch, alpha, sigma, means, emb,
               pof, eof, cond,
               alpha_v, sigma_v, mean_v, idx_v, rows_v, pbuf, ebuf,
               pobuf, eobuf, idx_t, rows_t, pbuf_t, ebuf_t, pobuf_t,
               eobuf_t, sem):
    wid = _worker_id()
    pltpu.sync_copy(alpha, alpha_v)
    pltpu.sync_copy(sigma, sigma_v)
    pltpu.sync_copy(means, mean_v)
    nb, rb = _chunk_of(wid)

    def blk(j, carry):
        r0 = rb + j * _BLK
        pltpu.sync_copy(batch.at[pl.ds(r0, _BLK)], idx_v)
        gather = pltpu.async_copy(emb.at[idx_v], rows_v, sem)
        pltpu.sync_copy(posf.at[pl.ds(r0 * 3, _BLK * 3)], pbuf)
        pltpu.sync_copy(epsf.at[pl.ds(r0 * 3, _BLK * 3)], ebuf)
        _center_block(_BLK, idx_v, pbuf, ebuf, alpha_v, sigma_v, mean_v,
                      pobuf, eobuf)
        pltpu.sync_copy(pobuf, pof.at[pl.ds(r0 * 3, _BLK * 3)])
        pltpu.sync_copy(eobuf, eof.at[pl.ds(r0 * 3, _BLK * 3)])
        gather.wait()
        pltpu.sync_copy(rows_v, cond.at[pl.ds(r0, _BLK), :])
        return carry

    lax.fori_loop(0, nb, blk, 0)

    @pl.when(wid == _NW - 1)
    def _tail():
        r0 = _TAIL_OFF
        pltpu.sync_copy(batch.at[pl.ds(r0, _TAIL)], idx_t)
        gather = pltpu.async_copy(emb.at[idx_t], rows_t, sem)
        pltpu.sync_copy(posf.at[pl.ds(r0 * 3, _TAIL * 3)], pbuf_t)
        pltpu.sync_copy(epsf.at[pl.ds(r0 * 3, _TAIL * 3)], ebuf_t)
        _center_block(_TAIL, idx_t, pbuf_t, ebuf_t, alpha_v, sigma_v,
                      mean_v, pobuf_t, eobuf_t)
        pltpu.sync_copy(pobuf_t, pof.at[pl.ds(r0 * 3, _TAIL * 3)])
        pltpu.sync_copy(eobuf_t, eof.at[pl.ds(r0 * 3, _TAIL * 3)])
        gather.wait()
        pltpu.sync_copy(rows_t, cond.at[pl.ds(r0, _TAIL), :])


def _sc_main(posf, epsf, batch, alpha, sigma, means, emb):
    mesh = plsc.VectorSubcoreMesh(core_axis_name="c", subcore_axis_name="s")
    f = pl.kernel(
        _main_body,
        out_type=(
            jax.ShapeDtypeStruct((_N * 3,), jnp.float32),
            jax.ShapeDtypeStruct((_N * 3,), jnp.float32),
            jax.ShapeDtypeStruct((_N, _D), jnp.float32),
        ),
        mesh=mesh,
        scratch_types=[
            pltpu.VMEM((_G,), jnp.float32),
            pltpu.VMEM((_G,), jnp.float32),
            pltpu.VMEM((_G, 8), jnp.float32),
            pltpu.VMEM((_BLK,), jnp.int32),
            pltpu.VMEM((_BLK, _D), jnp.float32),
            pltpu.VMEM((_BLK * 3,), jnp.float32),
            pltpu.VMEM((_BLK * 3,), jnp.float32),
            pltpu.VMEM((_BLK * 3,), jnp.float32),
            pltpu.VMEM((_BLK * 3,), jnp.float32),
            pltpu.VMEM((_TAIL,), jnp.int32),
            pltpu.VMEM((_TAIL, _D), jnp.float32),
            pltpu.VMEM((_TAIL * 3,), jnp.float32),
            pltpu.VMEM((_TAIL * 3,), jnp.float32),
            pltpu.VMEM((_TAIL * 3,), jnp.float32),
            pltpu.VMEM((_TAIL * 3,), jnp.float32),
            pltpu.SemaphoreType.DMA,
        ],
    )
    return f(posf, epsf, batch, alpha, sigma, means, emb)


def kernel(pos, eps, batch, t):
    posf = pos.reshape(-1)
    epsf = eps.reshape(-1)
    t2 = t.reshape(_G, 1)
    alpha, sigma, emb = _tables(t2)
    partials = _sc_sums(posf, epsf, batch)
    means = _combine(partials)
    pof, eof, cond = _sc_main(posf, epsf, batch, alpha.reshape(-1),
                              sigma.reshape(-1), means, emb)
    return (pof.reshape(_N, 3), eof.reshape(_N, 3), cond, alpha, sigma)


# speedup vs baseline: 3.4431x; 3.4431x over previous
import functools
import math

import jax
import jax.numpy as jnp
from jax import lax
from jax.experimental import pallas as pl
from jax.experimental.pallas import tpu as pltpu
import jax.experimental.pallas.tpu_sc as plsc

_N = 100000
_G = 1024
_T_MAX = 1000
_D = 128
_HALF = _D // 2

_NC = 2
_NS = 16
_NW = _NC * _NS
_BLK = 128
_NBLK = _N // _BLK
_TAIL = _N - _NBLK * _BLK
_EXTRA = _NBLK - 24 * _NW
_TAIL_OFF = _NBLK * _BLK


def _tables_body(t_ref, alpha_ref, sigma_ref, emb_ref):
    tf = t_ref[...].astype(jnp.float32)
    ang = tf * (0.5 * math.pi / _T_MAX)
    alpha_ref[...] = jnp.cos(ang)
    sigma_ref[...] = jnp.sin(ang)
    j = lax.broadcasted_iota(jnp.int32, (_G, _HALF), 1).astype(jnp.float32)
    freqs = jnp.exp(j * (-math.log(10000.0) / _HALF))
    args = tf * freqs
    emb_ref[:, :_HALF] = jnp.sin(args)
    emb_ref[:, _HALF:] = jnp.cos(args)


def _tables(t2):
    return pl.pallas_call(
        _tables_body,
        out_shape=(
            jax.ShapeDtypeStruct((_G, 1), jnp.float32),
            jax.ShapeDtypeStruct((_G, 1), jnp.float32),
            jax.ShapeDtypeStruct((_G, _D), jnp.float32),
        ),
    )(t2)


def _combine_body(p_ref, means_ref):
    p = p_ref[0] + p_ref[1]
    cnt = jnp.maximum(p[:, 6:7], 1.0)
    means_ref[...] = p / cnt


def _combine(partials):
    return pl.pallas_call(
        _combine_body,
        out_shape=jax.ShapeDtypeStruct((_G, 8), jnp.float32),
    )(partials)


def _worker_id():
    return lax.axis_index("s") * _NC + lax.axis_index("c")


def _chunk_of(wid):
    nb = 24 + jnp.where(wid < _EXTRA, 1, 0)
    base_blk = 24 * wid + jnp.minimum(wid, _EXTRA)
    return nb, base_blk * _BLK


def _iota16():
    return lax.iota(jnp.int32, 16)


def _fill_vals(n, idx_v, pbuf, ebuf, vals):
    iota = _iota16()
    ones = jnp.ones((16,), jnp.float32)
    for k in range(n // 16):
        rv = iota + k * 16
        rv3 = rv * 3
        for base, buf in ((0, pbuf), (3, ebuf)):
            for c in range(3):
                v = plsc.load_gather(buf, [rv3 + c])
                plsc.store_scatter(vals, [rv, jnp.full((16,), base + c, jnp.int32)], v)
        plsc.store_scatter(vals, [rv, jnp.full((16,), 6, jnp.int32)], ones)


def _zero_col7(n, vals):
    iota = _iota16()
    zeros = jnp.zeros((16,), jnp.float32)
    for k in range(n // 16):
        rv = iota + k * 16
        plsc.store_scatter(vals, [rv, jnp.full((16,), 7, jnp.int32)], zeros)


def _sums_body(posf, epsf, batch, out, acc, idx_v, pbuf, ebuf, vals,
               idx_t, pbuf_t, ebuf_t, vals_t, zbuf, obuf):
    c = lax.axis_index("c")
    s = lax.axis_index("s")
    wid = _worker_id()

    iota = _iota16()
    zeros = jnp.zeros((16,), jnp.float32)
    for i in range(32):
        rv = i * 2 + iota // 8
        cv = iota % 8
        plsc.store_scatter(zbuf, [rv, cv], zeros)
    pltpu.sync_copy(zbuf, acc.at[pl.ds(s * 64, 64), :])
    plsc.subcore_barrier()

    _zero_col7(_BLK, vals)
    nb, rb = _chunk_of(wid)

    def blk(j, carry):
        r0 = rb + j * _BLK
        pltpu.sync_copy(batch.at[pl.ds(r0, _BLK)], idx_v)
        pltpu.sync_copy(posf.at[pl.ds(r0 * 3, _BLK * 3)], pbuf)
        pltpu.sync_copy(epsf.at[pl.ds(r0 * 3, _BLK * 3)], ebuf)
        _fill_vals(_BLK, idx_v, pbuf, ebuf, vals)
        pltpu.sync_copy(vals, acc.at[idx_v], add=True)
        return carry

    lax.fori_loop(0, nb, blk, 0)

    @pl.when(wid == _NW - 1)
    def _tail():
        r0 = _TAIL_OFF
        pltpu.sync_copy(batch.at[pl.ds(r0, _TAIL)], idx_t)
        pltpu.sync_copy(posf.at[pl.ds(r0 * 3, _TAIL * 3)], pbuf_t)
        pltpu.sync_copy(epsf.at[pl.ds(r0 * 3, _TAIL * 3)], ebuf_t)
        _zero_col7(_TAIL, vals_t)
        _fill_vals(_TAIL, idx_t, pbuf_t, ebuf_t, vals_t)
        pltpu.sync_copy(vals_t, acc.at[idx_t], add=True)

    plsc.subcore_barrier()
    pltpu.sync_copy(acc.at[pl.ds(s * 64, 64), :], obuf)
    pltpu.sync_copy(obuf, out.at[c, pl.ds(s * 64, 64), :])


def _sc_sums(posf, epsf, batch):
    mesh = plsc.VectorSubcoreMesh(core_axis_name="c", subcore_axis_name="s")
    f = pl.kernel(
        _sums_body,
        out_type=jax.ShapeDtypeStruct((_NC, _G, 8), jnp.float32),
        mesh=mesh,
        compiler_params=pltpu.CompilerParams(needs_layout_passes=False, use_tc_tiling_on_sc=False),
        scratch_types=[
            pltpu.VMEM_SHARED((_G, 8), jnp.float32),
            pltpu.VMEM((_BLK,), jnp.int32),
            pltpu.VMEM((_BLK * 3,), jnp.float32),
            pltpu.VMEM((_BLK * 3,), jnp.float32),
            pltpu.VMEM((_BLK, 8), jnp.float32),
            pltpu.VMEM((_TAIL,), jnp.int32),
            pltpu.VMEM((_TAIL * 3,), jnp.float32),
            pltpu.VMEM((_TAIL * 3,), jnp.float32),
            pltpu.VMEM((_TAIL, 8), jnp.float32),
            pltpu.VMEM((64, 8), jnp.float32),
            pltpu.VMEM((64, 8), jnp.float32),
        ],
    )
    return f(posf, epsf, batch)


def _center_block(n, idx_v, pbuf, ebuf, alpha_v, sigma_v, mean_v, pobuf, eobuf):
    iota = _iota16()
    for k in range(n // 16):
        rv = iota + k * 16
        rv3 = rv * 3
        g = idx_v[pl.ds(k * 16, 16)]
        a = plsc.load_gather(alpha_v, [g])
        sg = plsc.load_gather(sigma_v, [g])
        for c in range(3):
            px = plsc.load_gather(pbuf, [rv3 + c])
            ex = plsc.load_gather(ebuf, [rv3 + c])
            mp = plsc.load_gather(mean_v, [g, jnp.full((16,), c, jnp.int32)])
            me = plsc.load_gather(mean_v, [g, jnp.full((16,), 3 + c, jnp.int32)])
            x = px - mp
            e = ex - me
            plsc.store_scatter(eobuf, [rv3 + c], e)
            plsc.store_scatter(pobuf, [rv3 + c], a * x + sg * e)


def _main_body(posf, epsf, batch, alpha, sigma, means, emb,
               pof, eof, cond,
               alpha_v, sigma_v, mean_v, idx_v, rows_v, pbuf, ebuf,
               pobuf, eobuf, idx_t, rows_t, pbuf_t, ebuf_t, pobuf_t,
               eobuf_t, sem):
    wid = _worker_id()
    pltpu.sync_copy(alpha, alpha_v)
    pltpu.sync_copy(sigma, sigma_v)
    pltpu.sync_copy(means, mean_v)
    nb, rb = _chunk_of(wid)

    def blk(j, carry):
        r0 = rb + j * _BLK
        pltpu.sync_copy(batch.at[pl.ds(r0, _BLK)], idx_v)
        gather = pltpu.async_copy(emb.at[idx_v], rows_v, sem)
        pltpu.sync_copy(posf.at[pl.ds(r0 * 3, _BLK * 3)], pbuf)
        pltpu.sync_copy(epsf.at[pl.ds(r0 * 3, _BLK * 3)], ebuf)
        _center_block(_BLK, idx_v, pbuf, ebuf, alpha_v, sigma_v, mean_v,
                      pobuf, eobuf)
        pltpu.sync_copy(pobuf, pof.at[pl.ds(r0 * 3, _BLK * 3)])
        pltpu.sync_copy(eobuf, eof.at[pl.ds(r0 * 3, _BLK * 3)])
        gather.wait()
        pltpu.sync_copy(rows_v, cond.at[pl.ds(r0, _BLK), :])
        return carry

    lax.fori_loop(0, nb, blk, 0)

    @pl.when(wid == _NW - 1)
    def _tail():
        r0 = _TAIL_OFF
        pltpu.sync_copy(batch.at[pl.ds(r0, _TAIL)], idx_t)
        gather = pltpu.async_copy(emb.at[idx_t], rows_t, sem)
        pltpu.sync_copy(posf.at[pl.ds(r0 * 3, _TAIL * 3)], pbuf_t)
        pltpu.sync_copy(epsf.at[pl.ds(r0 * 3, _TAIL * 3)], ebuf_t)
        _center_block(_TAIL, idx_t, pbuf_t, ebuf_t, alpha_v, sigma_v,
                      mean_v, pobuf_t, eobuf_t)
        pltpu.sync_copy(pobuf_t, pof.at[pl.ds(r0 * 3, _TAIL * 3)])
        pltpu.sync_copy(eobuf_t, eof.at[pl.ds(r0 * 3, _TAIL * 3)])
        gather.wait()
        pltpu.sync_copy(rows_t, cond.at[pl.ds(r0, _TAIL), :])


def _sc_main(posf, epsf, batch, alpha, sigma, means, emb):
    mesh = plsc.VectorSubcoreMesh(core_axis_name="c", subcore_axis_name="s")
    f = pl.kernel(
        _main_body,
        out_type=(
            jax.ShapeDtypeStruct((_N * 3,), jnp.float32),
            jax.ShapeDtypeStruct((_N * 3,), jnp.float32),
            jax.ShapeDtypeStruct((_N, _D), jnp.float32),
        ),
        mesh=mesh,
        compiler_params=pltpu.CompilerParams(needs_layout_passes=False, use_tc_tiling_on_sc=False),
        scratch_types=[
            pltpu.VMEM((_G,), jnp.float32),
            pltpu.VMEM((_G,), jnp.float32),
            pltpu.VMEM((_G, 8), jnp.float32),
            pltpu.VMEM((_BLK,), jnp.int32),
            pltpu.VMEM((_BLK, _D), jnp.float32),
            pltpu.VMEM((_BLK * 3,), jnp.float32),
            pltpu.VMEM((_BLK * 3,), jnp.float32),
            pltpu.VMEM((_BLK * 3,), jnp.float32),
            pltpu.VMEM((_BLK * 3,), jnp.float32),
            pltpu.VMEM((_TAIL,), jnp.int32),
            pltpu.VMEM((_TAIL, _D), jnp.float32),
            pltpu.VMEM((_TAIL * 3,), jnp.float32),
            pltpu.VMEM((_TAIL * 3,), jnp.float32),
            pltpu.VMEM((_TAIL * 3,), jnp.float32),
            pltpu.VMEM((_TAIL * 3,), jnp.float32),
            pltpu.SemaphoreType.DMA,
        ],
    )
    return f(posf, epsf, batch, alpha, sigma, means, emb)


def kernel(pos, eps, batch, t):
    posf = pos.reshape(-1)
    epsf = eps.reshape(-1)
    t2 = t.reshape(_G, 1)
    alpha, sigma, emb = _tables(t2)
    partials = _sc_sums(posf, epsf, batch)
    means = _combine(partials)
    pof, eof, cond = _sc_main(posf, epsf, batch, alpha.reshape(-1),
                              sigma.reshape(-1), means, emb)
    return (pof.reshape(_N, 3), eof.reshape(_N, 3), cond, alpha, sigma)


# speedup vs baseline: 3.6446x; 1.0585x over previous
import math

import jax
import jax.numpy as jnp
from jax import lax
from jax.experimental import pallas as pl
from jax.experimental.pallas import tpu as pltpu
import jax.experimental.pallas.tpu_sc as plsc

_N = 100000
_G = 1024
_T_MAX = 1000
_D = 128
_HALF = _D // 2

_NC = 2
_NS = 16
_NW = _NC * _NS
_BLK = 128
_NBLK = _N // _BLK
_TAIL = _N - _NBLK * _BLK
_EXTRA = _NBLK - 24 * _NW
_MAXB = 25
_TAIL_OFF = _NBLK * _BLK
_BA = 2000

_SC_PARAMS = pltpu.CompilerParams(needs_layout_passes=False,
                                  use_tc_tiling_on_sc=False)


def _pack_body(p_ref, e_ref, v_ref):
    v_ref[:, 0:3] = p_ref[...]
    v_ref[:, 3:6] = e_ref[...]
    v_ref[:, 6:7] = jnp.ones((_BA, 1), jnp.float32)
    v_ref[:, 7:8] = jnp.zeros((_BA, 1), jnp.float32)


def _pack(pos, eps):
    return pl.pallas_call(
        _pack_body,
        grid=(_N // _BA,),
        in_specs=[
            pl.BlockSpec((_BA, 3), lambda i: (i, 0)),
            pl.BlockSpec((_BA, 3), lambda i: (i, 0)),
        ],
        out_specs=pl.BlockSpec((_BA, 8), lambda i: (i, 0)),
        out_shape=jax.ShapeDtypeStruct((_N, 8), jnp.float32),
    )(pos, eps)


def _tables_body(t_ref, alpha_ref, sigma_ref, emb_ref):
    tf = t_ref[...].astype(jnp.float32)
    ang = tf * (0.5 * math.pi / _T_MAX)
    alpha_ref[...] = jnp.cos(ang)
    sigma_ref[...] = jnp.sin(ang)
    j = lax.broadcasted_iota(jnp.int32, (_G, _HALF), 1).astype(jnp.float32)
    freqs = jnp.exp(j * (-math.log(10000.0) / _HALF))
    args = tf * freqs
    emb_ref[:, :_HALF] = jnp.sin(args)
    emb_ref[:, _HALF:] = jnp.cos(args)


def _tables(t2):
    return pl.pallas_call(
        _tables_body,
        out_shape=(
            jax.ShapeDtypeStruct((_G, 1), jnp.float32),
            jax.ShapeDtypeStruct((_G, 1), jnp.float32),
            jax.ShapeDtypeStruct((_G, _D), jnp.float32),
        ),
    )(t2)


def _combine_body(p_ref, a_ref, s_ref, t8_ref):
    p = p_ref[0] + p_ref[1]
    cnt = jnp.maximum(p[:, 6:7], 1.0)
    t8_ref[:, 0:6] = p[:, 0:6] / cnt
    t8_ref[:, 6:7] = a_ref[...]
    t8_ref[:, 7:8] = s_ref[...]


def _combine(partials, alpha, sigma):
    return pl.pallas_call(
        _combine_body,
        out_shape=jax.ShapeDtypeStruct((_G, 8), jnp.float32),
    )(partials, alpha, sigma)


def _final_body(p_ref, e_ref, g_ref, po_ref, eo_ref):
    g = g_ref[...]
    x = p_ref[...] - g[:, 0:3]
    e = e_ref[...] - g[:, 3:6]
    eo_ref[...] = e
    po_ref[...] = g[:, 6:7] * x + g[:, 7:8] * e


def _final(pos, eps, g8):
    return pl.pallas_call(
        _final_body,
        grid=(_N // _BA,),
        in_specs=[
            pl.BlockSpec((_BA, 3), lambda i: (i, 0)),
            pl.BlockSpec((_BA, 3), lambda i: (i, 0)),
            pl.BlockSpec((_BA, 8), lambda i: (i, 0)),
        ],
        out_specs=[
            pl.BlockSpec((_BA, 3), lambda i: (i, 0)),
            pl.BlockSpec((_BA, 3), lambda i: (i, 0)),
        ],
        out_shape=(
            jax.ShapeDtypeStruct((_N, 3), jnp.float32),
            jax.ShapeDtypeStruct((_N, 3), jnp.float32),
        ),
    )(pos, eps, g8)


def _worker_id():
    return lax.axis_index("s") * _NC + lax.axis_index("c")


def _chunk_of(wid):
    nb = 24 + jnp.where(wid < _EXTRA, 1, 0)
    base_blk = 24 * wid + jnp.minimum(wid, _EXTRA)
    return nb, base_blk * _BLK


def _iota16():
    return lax.iota(jnp.int32, 16)


def _mesh():
    return plsc.VectorSubcoreMesh(core_axis_name="c", subcore_axis_name="s")


def _sums_body(batch, vals8, out, acc, idxb, vbuf, idx_t, vbuf_t, zbuf, obuf):
    c = lax.axis_index("c")
    s = lax.axis_index("s")
    wid = _worker_id()

    iota = _iota16()
    zeros = jnp.zeros((16,), jnp.float32)
    for i in range(32):
        rv = i * 2 + iota // 8
        cv = iota % 8
        plsc.store_scatter(zbuf, [rv, cv], zeros)
    pltpu.sync_copy(zbuf, acc.at[pl.ds(s * 64, 64), :])
    plsc.subcore_barrier()

    nb, rb = _chunk_of(wid)

    def blk(j, carry):
        r0 = rb + j * _BLK
        pltpu.sync_copy(batch.at[pl.ds(r0, _BLK)], idxb)
        pltpu.sync_copy(vals8.at[pl.ds(r0, _BLK), :], vbuf)
        pltpu.sync_copy(vbuf, acc.at[idxb], add=True)
        return carry

    lax.fori_loop(0, nb, blk, 0)

    @pl.when(wid == _NW - 1)
    def _tail():
        r0 = _TAIL_OFF
        pltpu.sync_copy(batch.at[pl.ds(r0, _TAIL)], idx_t)
        pltpu.sync_copy(vals8.at[pl.ds(r0, _TAIL), :], vbuf_t)
        pltpu.sync_copy(vbuf_t, acc.at[idx_t], add=True)

    plsc.subcore_barrier()
    pltpu.sync_copy(acc.at[pl.ds(s * 64, 64), :], obuf)
    pltpu.sync_copy(obuf, out.at[c, pl.ds(s * 64, 64), :])


def _sc_sums(batch, vals8):
    f = pl.kernel(
        _sums_body,
        out_type=jax.ShapeDtypeStruct((_NC, _G, 8), jnp.float32),
        mesh=_mesh(),
        compiler_params=_SC_PARAMS,
        scratch_types=[
            pltpu.VMEM_SHARED((_G, 8), jnp.float32),
            pltpu.VMEM((_BLK,), jnp.int32),
            pltpu.VMEM((_BLK, 8), jnp.float32),
            pltpu.VMEM((_TAIL,), jnp.int32),
            pltpu.VMEM((_TAIL, 8), jnp.float32),
            pltpu.VMEM((64, 8), jnp.float32),
            pltpu.VMEM((64, 8), jnp.float32),
        ],
    )
    return f(batch, vals8)


def _g8_body(batch, table8, g8, idxg, gbuf, idx_t, gbuf_t, sem):
    wid = _worker_id()
    nb, rb = _chunk_of(wid)

    def blk(j, carry):
        r0 = rb + j * _BLK
        pltpu.sync_copy(batch.at[pl.ds(r0, _BLK)], idxg)
        pltpu.async_copy(table8.at[idxg], gbuf, sem).wait()
        pltpu.sync_copy(gbuf, g8.at[pl.ds(r0, _BLK), :])
        return carry

    lax.fori_loop(0, nb, blk, 0)

    @pl.when(wid == _NW - 1)
    def _tail():
        r0 = _TAIL_OFF
        pltpu.sync_copy(batch.at[pl.ds(r0, _TAIL)], idx_t)
        pltpu.async_copy(table8.at[idx_t], gbuf_t, sem).wait()
        pltpu.sync_copy(gbuf_t, g8.at[pl.ds(r0, _TAIL), :])


def _sc_g8(batch, table8):
    f = pl.kernel(
        _g8_body,
        out_type=jax.ShapeDtypeStruct((_N, 8), jnp.float32),
        mesh=_mesh(),
        compiler_params=_SC_PARAMS,
        scratch_types=[
            pltpu.VMEM((_BLK,), jnp.int32),
            pltpu.VMEM((_BLK, 8), jnp.float32),
            pltpu.VMEM((_TAIL,), jnp.int32),
            pltpu.VMEM((_TAIL, 8), jnp.float32),
            pltpu.SemaphoreType.DMA,
        ],
    )
    return f(batch, table8)


def _cond_body(batch, emb, cond, idxA, idxB, rowsA, rowsB,
               idx_t, rows_t, semGA, semGB, semWA, semWB, semT):
    wid = _worker_id()
    nb, rb = _chunk_of(wid)

    bufs = ((idxA, rowsA, semGA, semWA), (idxB, rowsB, semGB, semWB))

    def _issue(j, idx, rows, semG):
        r0 = rb + j * _BLK
        pltpu.sync_copy(batch.at[pl.ds(r0, _BLK)], idx)
        pltpu.async_copy(emb.at[idx], rows, semG)

    def _finish(j, idx, rows, semG, semW):
        r0 = rb + j * _BLK
        pltpu.make_async_copy(emb.at[idx], rows, semG).wait()
        pltpu.async_copy(rows, cond.at[pl.ds(r0, _BLK), :], semW)

    def _wait_write(j, idx, rows, semW):
        r0 = rb + j * _BLK
        pltpu.make_async_copy(rows, cond.at[pl.ds(r0, _BLK), :], semW).wait()

    def blk(j, carry):
        for par in range(2):
            @pl.when(jnp.logical_and(j % 2 == par, j < nb))
            def _do():
                idx, rows, semG, semW = bufs[par]

                @pl.when(j >= 2)
                def _w():
                    _wait_write(j - 2, idx, rows, semW)

                _issue(j, idx, rows, semG)

            @pl.when(jnp.logical_and(j % 2 == par, jnp.logical_and(j >= 1, j <= nb)))
            def _fin():
                oidx, orows, osemG, osemW = bufs[1 - par]
                _finish(j - 1, oidx, orows, osemG, osemW)

        return carry

    lax.fori_loop(0, _MAXB + 1, blk, 0)

    for par in range(2):
        @pl.when(jnp.logical_and((nb - 1) % 2 == par, nb >= 1))
        def _dr():
            idx, rows, semG, semW = bufs[par]
            _wait_write(nb - 1, idx, rows, semW)

        @pl.when(jnp.logical_and((nb - 2) % 2 == par, nb >= 2))
        def _dr2():
            idx, rows, semG, semW = bufs[par]
            _wait_write(nb - 2, idx, rows, semW)

    @pl.when(wid == _NW - 1)
    def _tail():
        r0 = _TAIL_OFF
        pltpu.sync_copy(batch.at[pl.ds(r0, _TAIL)], idx_t)
        pltpu.async_copy(emb.at[idx_t], rows_t, semT).wait()
        pltpu.sync_copy(rows_t, cond.at[pl.ds(r0, _TAIL), :])


def _sc_cond(batch, emb):
    f = pl.kernel(
        _cond_body,
        out_type=jax.ShapeDtypeStruct((_N, _D), jnp.float32),
        mesh=_mesh(),
        compiler_params=_SC_PARAMS,
        scratch_types=[
            pltpu.VMEM((_BLK,), jnp.int32),
            pltpu.VMEM((_BLK,), jnp.int32),
            pltpu.VMEM((_BLK, _D), jnp.float32),
            pltpu.VMEM((_BLK, _D), jnp.float32),
            pltpu.VMEM((_TAIL,), jnp.int32),
            pltpu.VMEM((_TAIL, _D), jnp.float32),
            pltpu.SemaphoreType.DMA,
            pltpu.SemaphoreType.DMA,
            pltpu.SemaphoreType.DMA,
            pltpu.SemaphoreType.DMA,
            pltpu.SemaphoreType.DMA,
        ],
    )
    return f(batch, emb)


def kernel(pos, eps, batch, t):
    t2 = t.reshape(_G, 1)
    vals8 = _pack(pos, eps)
    alpha, sigma, emb = _tables(t2)
    partials = _sc_sums(batch, vals8)
    table8 = _combine(partials, alpha, sigma)
    g8 = _sc_g8(batch, table8)
    emb2, g8 = lax.optimization_barrier((emb, g8))
    cond = _sc_cond(batch, emb2)
    pos_out, eps_c = _final(pos, eps, g8)
    return (pos_out, eps_c, cond, alpha, sigma)


# speedup vs baseline: 4.1889x; 1.1493x over previous
import math

import jax
import jax.numpy as jnp
from jax import lax
from jax.experimental import pallas as pl
from jax.experimental.pallas import tpu as pltpu
import jax.experimental.pallas.tpu_sc as plsc

_N = 100000
_G = 1024
_T_MAX = 1000
_D = 128
_HALF = _D // 2

_NC = 2
_NS = 16
_NW = _NC * _NS
_BLK = 128
_NBLK = _N // _BLK
_TAIL = _N - _NBLK * _BLK
_EXTRA = _NBLK - 24 * _NW
_MAXB = 25
_CH = _MAXB * _BLK
_TAIL_OFF = _NBLK * _BLK

_SC_PARAMS = pltpu.CompilerParams(needs_layout_passes=False,
                                  use_tc_tiling_on_sc=False)


def _tables_body(t_ref, alpha_ref, sigma_ref, emb_ref):
    tf = t_ref[...].astype(jnp.float32)
    ang = tf * (0.5 * math.pi / _T_MAX)
    alpha_ref[...] = jnp.cos(ang)
    sigma_ref[...] = jnp.sin(ang)
    j = lax.broadcasted_iota(jnp.int32, (_G, _HALF), 1).astype(jnp.float32)
    freqs = jnp.exp(j * (-math.log(10000.0) / _HALF))
    args = tf * freqs
    emb_ref[:, :_HALF] = jnp.sin(args)
    emb_ref[:, _HALF:] = jnp.cos(args)


def _tables(t2):
    return pl.pallas_call(
        _tables_body,
        out_shape=(
            jax.ShapeDtypeStruct((_G, 1), jnp.float32),
            jax.ShapeDtypeStruct((_G, 1), jnp.float32),
            jax.ShapeDtypeStruct((_G, _D), jnp.float32),
        ),
    )(t2)


def _combine_body(p_ref, a_ref, s_ref, t8_ref):
    p = p_ref[0] + p_ref[1]
    cnt = jnp.maximum(p[:, 6:7], 1.0)
    t8_ref[:, 0:6] = p[:, 0:6] / cnt
    t8_ref[:, 6:7] = a_ref[...]
    t8_ref[:, 7:8] = s_ref[...]


def _combine(partials, alpha, sigma):
    return pl.pallas_call(
        _combine_body,
        out_shape=jax.ShapeDtypeStruct((_G, 8), jnp.float32),
    )(partials, alpha, sigma)


def _worker_id():
    return lax.axis_index("s") * _NC + lax.axis_index("c")


def _chunk_of(wid):
    nb = 24 + jnp.where(wid < _EXTRA, 1, 0)
    base_blk = 24 * wid + jnp.minimum(wid, _EXTRA)
    return nb, base_blk * _BLK


def _iota16():
    return lax.iota(jnp.int32, 16)


def _mesh():
    return plsc.VectorSubcoreMesh(core_axis_name="c", subcore_axis_name="s")


def _full16(v):
    return jnp.full((16,), v, jnp.int32)


def _copy_chunk_in(src2d, dst, rb, nb):
    @pl.when(nb == _MAXB)
    def _a():
        pltpu.sync_copy(src2d.at[pl.ds(rb, _MAXB * _BLK), :], dst)

    @pl.when(nb == _MAXB - 1)
    def _b():
        pltpu.sync_copy(src2d.at[pl.ds(rb, (_MAXB - 1) * _BLK), :],
                        dst.at[pl.ds(0, (_MAXB - 1) * _BLK), :])


def _copy_chunk_out(src, dst2d, rb, nb):
    @pl.when(nb == _MAXB)
    def _a():
        pltpu.sync_copy(src, dst2d.at[pl.ds(rb, _MAXB * _BLK), :])

    @pl.when(nb == _MAXB - 1)
    def _b():
        pltpu.sync_copy(src.at[pl.ds(0, (_MAXB - 1) * _BLK), :],
                        dst2d.at[pl.ds(rb, (_MAXB - 1) * _BLK), :])


def _build_vals(k0, n16, ichunk, pchunk, echunk, vals):
    iota = _iota16()
    ones = jnp.ones((16,), jnp.float32)
    for k in range(n16):
        lrow = (k0 + k) * 16 + iota
        vrow = k * 16 + iota
        for base, buf in ((0, pchunk), (3, echunk)):
            for c in range(3):
                v = plsc.load_gather(buf, [lrow, _full16(c)])
                plsc.store_scatter(vals, [vrow, _full16(base + c)], v)
        plsc.store_scatter(vals, [vrow, _full16(6)], ones)


def _zero_col7(n16, vals):
    iota = _iota16()
    zeros = jnp.zeros((16,), jnp.float32)
    for k in range(n16):
        plsc.store_scatter(vals, [k * 16 + iota, _full16(7)], zeros)


def _sums_body(pos, eps, batch, out, acc, pchunk, echunk,
               idx0, idx1, vals0, vals1, ptail, etail, idx_t, vals_t,
               zbuf, obuf, sem0, sem1):
    c = lax.axis_index("c")
    s = lax.axis_index("s")
    wid = _worker_id()

    iota = _iota16()
    zeros = jnp.zeros((16,), jnp.float32)
    for i in range(32):
        plsc.store_scatter(zbuf, [i * 2 + iota // 8, iota % 8], zeros)
    pltpu.sync_copy(zbuf, acc.at[pl.ds(s * 64, 64), :])
    plsc.subcore_barrier()

    nb, rb = _chunk_of(wid)
    _copy_chunk_in(pos, pchunk, rb, nb)
    _copy_chunk_in(eps, echunk, rb, nb)
    _zero_col7(8, vals0)
    _zero_col7(8, vals1)

    def blk(j, carry):
        pltpu.sync_copy(batch.at[pl.ds(rb + j * _BLK, _BLK)], idx0)
        iota = _iota16()
        ones = jnp.ones((16,), jnp.float32)
        for k in range(8):
            lrow = j * _BLK + k * 16 + iota
            vrow = k * 16 + iota
            for base, buf in ((0, pchunk), (3, echunk)):
                for c in range(3):
                    v = plsc.load_gather(buf, [lrow, _full16(c)])
                    plsc.store_scatter(vals0, [vrow, _full16(base + c)], v)
            plsc.store_scatter(vals0, [vrow, _full16(6)], ones)
        pltpu.sync_copy(vals0, acc.at[idx0], add=True)
        return carry

    lax.fori_loop(0, nb, blk, 0)

    @pl.when(wid == _NW - 1)
    def _tail():
        r0 = _TAIL_OFF
        pltpu.sync_copy(batch.at[pl.ds(r0, _TAIL)], idx_t)
        pltpu.sync_copy(pos.at[pl.ds(r0, _TAIL), :], ptail)
        pltpu.sync_copy(eps.at[pl.ds(r0, _TAIL), :], etail)
        _zero_col7(2, vals_t)
        _build_vals(0, 2, idx_t, ptail, etail, vals_t)
        pltpu.sync_copy(vals_t, acc.at[idx_t], add=True)

    plsc.subcore_barrier()
    pltpu.sync_copy(acc.at[pl.ds(s * 64, 64), :], obuf)
    pltpu.sync_copy(obuf, out.at[c, pl.ds(s * 64, 64), :])


def _sc_sums(pos, eps, batch):
    f = pl.kernel(
        _sums_body,
        out_type=jax.ShapeDtypeStruct((_NC, _G, 8), jnp.float32),
        mesh=_mesh(),
        compiler_params=_SC_PARAMS,
        scratch_types=[
            pltpu.VMEM_SHARED((_G, 8), jnp.float32),
            pltpu.VMEM((_CH, 3), jnp.float32),
            pltpu.VMEM((_CH, 3), jnp.float32),
            pltpu.VMEM((_BLK,), jnp.int32),
            pltpu.VMEM((_BLK,), jnp.int32),
            pltpu.VMEM((_BLK, 8), jnp.float32),
            pltpu.VMEM((_BLK, 8), jnp.float32),
            pltpu.VMEM((_TAIL, 3), jnp.float32),
            pltpu.VMEM((_TAIL, 3), jnp.float32),
            pltpu.VMEM((_TAIL,), jnp.int32),
            pltpu.VMEM((_TAIL, 8), jnp.float32),
            pltpu.VMEM((64, 8), jnp.float32),
            pltpu.VMEM((64, 8), jnp.float32),
            pltpu.SemaphoreType.DMA,
            pltpu.SemaphoreType.DMA,
        ],
    )
    return f(pos, eps, batch)


def _center_rows(n16, k0, ichunk_vals, pchunk, echunk, t8v, pochunk, eochunk):
    iota = _iota16()
    for k in range(n16):
        lrow = (k0 + k) * 16 + iota
        g = plsc.load_gather(ichunk_vals, [lrow])
        a = plsc.load_gather(t8v, [g, _full16(6)])
        sg = plsc.load_gather(t8v, [g, _full16(7)])
        for c in range(3):
            px = plsc.load_gather(pchunk, [lrow, _full16(c)])
            ex = plsc.load_gather(echunk, [lrow, _full16(c)])
            mp = plsc.load_gather(t8v, [g, _full16(c)])
            me = plsc.load_gather(t8v, [g, _full16(3 + c)])
            x = px - mp
            e = ex - me
            plsc.store_scatter(eochunk, [lrow, _full16(c)], e)
            plsc.store_scatter(pochunk, [lrow, _full16(c)], a * x + sg * e)


def _center_body(pos, eps, batch, table8, po, eo, t8v, ichunk,
                 pchunk, echunk, pochunk, eochunk,
                 ptail, etail, idx_t, potail, eotail):
    wid = _worker_id()
    nb, rb = _chunk_of(wid)
    pltpu.sync_copy(table8, t8v)
    pltpu.sync_copy(batch.at[pl.ds(rb, _CH - _BLK)], ichunk.at[pl.ds(0, _CH - _BLK)])

    @pl.when(nb == _MAXB)
    def _i25():
        pltpu.sync_copy(batch.at[pl.ds(rb + _CH - _BLK, _BLK)],
                        ichunk.at[pl.ds(_CH - _BLK, _BLK)])

    _copy_chunk_in(pos, pchunk, rb, nb)
    _copy_chunk_in(eps, echunk, rb, nb)
    for j in range(_MAXB):
        @pl.when(j < nb)
        def _do():
            _center_rows(8, j * 8, ichunk, pchunk, echunk, t8v,
                         pochunk, eochunk)
    _copy_chunk_out(pochunk, po, rb, nb)
    _copy_chunk_out(eochunk, eo, rb, nb)

    @pl.when(wid == _NW - 1)
    def _tail():
        r0 = _TAIL_OFF
        pltpu.sync_copy(batch.at[pl.ds(r0, _TAIL)], idx_t)
        pltpu.sync_copy(pos.at[pl.ds(r0, _TAIL), :], ptail)
        pltpu.sync_copy(eps.at[pl.ds(r0, _TAIL), :], etail)
        _center_rows(2, 0, idx_t, ptail, etail, t8v, potail, eotail)
        pltpu.sync_copy(potail, po.at[pl.ds(r0, _TAIL), :])
        pltpu.sync_copy(eotail, eo.at[pl.ds(r0, _TAIL), :])


def _sc_center(pos, eps, batch, table8):
    f = pl.kernel(
        _center_body,
        out_type=(
            jax.ShapeDtypeStruct((_N, 3), jnp.float32),
            jax.ShapeDtypeStruct((_N, 3), jnp.float32),
        ),
        mesh=_mesh(),
        compiler_params=_SC_PARAMS,
        scratch_types=[
            pltpu.VMEM((_G, 8), jnp.float32),
            pltpu.VMEM((_CH,), jnp.int32),
            pltpu.VMEM((_CH, 3), jnp.float32),
            pltpu.VMEM((_CH, 3), jnp.float32),
            pltpu.VMEM((_CH, 3), jnp.float32),
            pltpu.VMEM((_CH, 3), jnp.float32),
            pltpu.VMEM((_TAIL, 3), jnp.float32),
            pltpu.VMEM((_TAIL, 3), jnp.float32),
            pltpu.VMEM((_TAIL,), jnp.int32),
            pltpu.VMEM((_TAIL, 3), jnp.float32),
            pltpu.VMEM((_TAIL, 3), jnp.float32),
        ],
    )
    return f(pos, eps, batch, table8)


_NBUF = 3


def _cond_body(batch, emb, cond, idxs, rows, semsG, semsW, idx_t, rows_t, semT):
    wid = _worker_id()
    nb, rb = _chunk_of(wid)

    def _gather(j, slot):
        return pltpu.make_async_copy(emb.at[idxs[slot]], rows[slot],
                                     semsG[slot])

    def _write(j, slot):
        return pltpu.make_async_copy(
            rows[slot], cond.at[pl.ds(rb + j * _BLK, _BLK), :], semsW[slot])

    for j in range(_MAXB + 2):
        slot = j % _NBUF
        if j < _MAXB:
            @pl.when(j < nb)
            def _issue():
                if j >= _NBUF:
                    _write(j - _NBUF, slot).wait()
                pltpu.sync_copy(batch.at[pl.ds(rb + j * _BLK, _BLK)],
                                idxs[slot])
                pltpu.async_copy(emb.at[idxs[slot]], rows[slot], semsG[slot])

        if j >= 2:
            jj = j - 2
            wslot = jj % _NBUF

            @pl.when(jj < nb)
            def _fin():
                _gather(jj, wslot).wait()
                pltpu.async_copy(rows[wslot],
                                 cond.at[pl.ds(rb + jj * _BLK, _BLK), :],
                                 semsW[wslot])

    for j in range(_MAXB - _NBUF - 1, _MAXB):
        slot = j % _NBUF

        @pl.when(jnp.logical_and(j < nb, j + _NBUF >= nb))
        def _drain():
            _write(j, slot).wait()

    @pl.when(wid == _NW - 1)
    def _tail():
        r0 = _TAIL_OFF
        pltpu.sync_copy(batch.at[pl.ds(r0, _TAIL)], idx_t)
        pltpu.async_copy(emb.at[idx_t], rows_t, semT).wait()
        pltpu.sync_copy(rows_t, cond.at[pl.ds(r0, _TAIL), :])


def _cond_body_wrap(batch, emb, cond,
                    idx0, idx1, idx2, rows0, rows1, rows2,
                    semG0, semG1, semG2, semW0, semW1, semW2,
                    idx_t, rows_t, semT):
    _cond_body(batch, emb, cond, (idx0, idx1, idx2), (rows0, rows1, rows2),
               (semG0, semG1, semG2), (semW0, semW1, semW2),
               idx_t, rows_t, semT)


def _sc_cond(batch, emb):
    f = pl.kernel(
        _cond_body_wrap,
        out_type=jax.ShapeDtypeStruct((_N, _D), jnp.float32),
        mesh=_mesh(),
        compiler_params=_SC_PARAMS,
        scratch_types=(
            [pltpu.VMEM((_BLK,), jnp.int32)] * 3
            + [pltpu.VMEM((_BLK, _D), jnp.float32)] * 3
            + [pltpu.SemaphoreType.DMA] * 6
            + [pltpu.VMEM((_TAIL,), jnp.int32),
               pltpu.VMEM((_TAIL, _D), jnp.float32),
               pltpu.SemaphoreType.DMA]
        ),
    )
    return f(batch, emb)


def kernel(pos, eps, batch, t):
    t2 = t.reshape(_G, 1)
    alpha, sigma, emb = _tables(t2)
    partials = _sc_sums(pos, eps, batch)
    table8 = _combine(partials, alpha, sigma)
    pos_out, eps_c = _sc_center(pos, eps, batch, table8)
    emb2, pos_out = lax.optimization_barrier((emb, pos_out))
    cond = _sc_cond(batch, emb2)
    return (pos_out, eps_c, cond, alpha, sigma)
